# baseline (speedup 1.0000x reference)
import jax
import jax.numpy as jnp
from jax import lax
from jax.experimental import pallas as pl
from jax.experimental.pallas import tpu as pltpu
from jax.experimental.pallas import tpu_sc as plsc

D_MODEL = 64
SCALE = 8.0
NC, NS = 2, 16
NW = NC * NS
ROWS = 4096 * 200
NBLK = ROWS // 128
BLK_PER_W = NBLK // NW
QUADS = BLK_PER_W // 4
NJ = 200
NI = 32
SKEW = 129


def _emb_body(ta, tb, xph, out, xall, r0a, r0b, r1a, r1b,
              wb0, wb1, wb2, wb3, gs0, gs1, ws0, ws1, ws2, ws3):
    wid = lax.axis_index("s") * NC + lax.axis_index("c")
    blk0 = wid * BLK_PER_W

    pltpu.sync_copy(xph.at[pl.ds(blk0, BLK_PER_W)], xall)

    iota = lax.iota(jnp.int32, 16)
    fvecs = [iota + 16 * g for g in range(4)]

    rows_bufs = ((r0a, r0b), (r1a, r1b))
    gsems = (gs0, gs1)
    wbs = (wb0, wb1, wb2, wb3)
    wsems = (ws0, ws1, ws2, ws3)

    def fire_block(r, rows_v, gsem):
        ra, rb = rows_v
        pltpu.async_copy(ta.at[xall.at[r]], ra, gsem)
        pltpu.async_copy(tb.at[xall.at[r]], rb, gsem)

    fire_block(0, rows_bufs[0], gs0)
    fire_block(1, rows_bufs[1], gs1)

    def quad_body(k, carry):
        for q in range(4):
            rows_v = rows_bufs[q % 2]
            gsem = gsems[q % 2]
            wb = wbs[q]
            wsem = wsems[q]
            r = 4 * k + q
            c = blk0 + r
            jb = lax.shift_right_logical(c, 8)
            ib = lax.bitwise_and(lax.shift_right_logical(c, 3), 31)
            js = lax.bitwise_and(c, 7)
            j = jb * 8 + js

            pltpu.make_async_copy(ta.at[pl.ds(0, 128)], rows_v[0], gsem).wait()
            pltpu.make_async_copy(tb.at[pl.ds(0, 128)], rows_v[1], gsem).wait()

            @pl.when(k >= 1)
            def _drain_writes():
                pltpu.make_async_copy(
                    ta.at[pl.ds(0, 128)], rows_v[0], wsem
                ).wait()
                pltpu.make_async_copy(
                    tb.at[pl.ds(0, 128)], rows_v[1], wsem
                ).wait()

            @plsc.parallel_loop(0, 128, unroll=4)
            def _transpose(l):
                lv = jnp.full((16,), l, jnp.int32)
                for g in range(4):
                    half = rows_v[g // 2]
                    vals = half[l, pl.ds(16 * (g % 2), 16)] * SCALE
                    plsc.store_scatter(wb, [fvecs[g], lv], vals)

            for fb in range(8):
                pltpu.async_copy(
                    wb.at[pl.ds(fb * 8, 8), pl.ds(0, 128)],
                    out.at[j, fb, ib],
                    wsem,
                )

            @pl.when(r < BLK_PER_W - 2)
            def _refill():
                fire_block(r + 2, rows_v, gsem)
        return carry

    lax.fori_loop(0, QUADS, quad_body, 0)

    for b in range(4):
        pltpu.make_async_copy(
            ta.at[pl.ds(0, 128)], rows_bufs[b % 2][0], wsems[b]
        ).wait()
        pltpu.make_async_copy(
            tb.at[pl.ds(0, 128)], rows_bufs[b % 2][1], wsems[b]
        ).wait()


_emb = pl.kernel(
    _emb_body,
    out_type=jax.ShapeDtypeStruct((NJ, 8, NI, 8, 128), jnp.float32),
    mesh=plsc.VectorSubcoreMesh(core_axis_name="c", subcore_axis_name="s"),
    scratch_types=[
        pltpu.VMEM((BLK_PER_W, 128), jnp.int32),
        pltpu.VMEM((128, D_MODEL // 2), jnp.float32),
        pltpu.VMEM((128, D_MODEL // 2), jnp.float32),
        pltpu.VMEM((128, D_MODEL // 2), jnp.float32),
        pltpu.VMEM((128, D_MODEL // 2), jnp.float32),
        pltpu.VMEM((D_MODEL, SKEW), jnp.float32),
        pltpu.VMEM((D_MODEL, SKEW), jnp.float32),
        pltpu.VMEM((D_MODEL, SKEW), jnp.float32),
        pltpu.VMEM((D_MODEL, SKEW), jnp.float32),
        pltpu.SemaphoreType.DMA,
        pltpu.SemaphoreType.DMA,
        pltpu.SemaphoreType.DMA,
        pltpu.SemaphoreType.DMA,
        pltpu.SemaphoreType.DMA,
        pltpu.SemaphoreType.DMA,
    ],
    compiler_params=pltpu.CompilerParams(
        use_tc_tiling_on_sc=False, needs_layout_passes=False
    ),
)


@jax.jit
def _run(x, table):
    xph = x.T.reshape(25, 8, NI, 128).transpose(0, 2, 1, 3).reshape(NBLK, 128)
    o = _emb(table[:, : D_MODEL // 2], table[:, D_MODEL // 2 :], xph)
    o = o.transpose(2, 4, 0, 1, 3)
    return o.reshape(4096, NJ, D_MODEL)


def kernel(x, table):
    return _run(x, table)


# speedup vs baseline: 2.0957x; 2.0957x over previous
import jax
import jax.numpy as jnp
from jax import lax
from jax.experimental import pallas as pl
from jax.experimental.pallas import tpu as pltpu
from jax.experimental.pallas import tpu_sc as plsc

D_MODEL = 64
SCALE = 8.0
NC, NS = 2, 16
NW = NC * NS
ROWS = 4096 * 200
NBLK = ROWS // 128
BLK_PER_W = NBLK // NW
QUADS = BLK_PER_W // 4
NJ = 200
NI = 32
SKEW = 129


def _emb_body(table, xph, out, xall, rows0, rows1, rows2, rows3,
              wb0, wb1, wb2, wb3, gs0, gs1, gs2, gs3,
              ws0, ws1, ws2, ws3):
    wid = lax.axis_index("s") * NC + lax.axis_index("c")
    blk0 = wid * BLK_PER_W

    pltpu.sync_copy(xph.at[pl.ds(blk0, BLK_PER_W)], xall)

    iota = lax.iota(jnp.int32, 16)
    fvecs = [iota + 16 * g for g in range(4)]

    rows_bufs = (rows0, rows1, rows2, rows3)
    gsems = (gs0, gs1, gs2, gs3)
    wbs = (wb0, wb1, wb2, wb3)
    wsems = (ws0, ws1, ws2, ws3)

    def fire_block(r, rows_v, gsem):
        pltpu.async_copy(table.at[xall.at[r]], rows_v, gsem)

    for _p in range(4):
        fire_block(_p, rows_bufs[_p], gsems[_p])

    def quad_body(k, carry):
        for q in range(4):
            rows_v = rows_bufs[q]
            gsem = gsems[q]
            wb = wbs[q]
            wsem = wsems[q]
            r = 4 * k + q
            c = blk0 + r
            jb = lax.shift_right_logical(c, 8)
            ib = lax.bitwise_and(lax.shift_right_logical(c, 3), 31)
            js = lax.bitwise_and(c, 7)
            j = jb * 8 + js

            pltpu.make_async_copy(table.at[pl.ds(0, 128)], rows_v, gsem).wait()

            @pl.when(k >= 1)
            def _drain_writes():
                pltpu.make_async_copy(
                    table.at[pl.ds(0, 128)], rows_v, wsem
                ).wait()

            @plsc.parallel_loop(0, 128, unroll=4)
            def _transpose(l):
                lv = jnp.full((16,), l, jnp.int32)
                for g in range(4):
                    vals = rows_v[l, pl.ds(16 * g, 16)] * SCALE
                    plsc.store_scatter(wb, [fvecs[g], lv], vals)

            for fb in range(8):
                pltpu.async_copy(
                    wb.at[pl.ds(fb * 8, 8), pl.ds(0, 128)],
                    out.at[j, fb, ib],
                    wsem,
                )

            @pl.when(r < BLK_PER_W - 4)
            def _refill():
                fire_block(r + 4, rows_v, gsem)
        return carry

    lax.fori_loop(0, QUADS, quad_body, 0)

    for b in range(4):
        pltpu.make_async_copy(
            table.at[pl.ds(0, 128)], rows_bufs[b], wsems[b]
        ).wait()


_emb = pl.kernel(
    _emb_body,
    out_type=jax.ShapeDtypeStruct((NJ, 8, NI, 8, 128), jnp.float32),
    mesh=plsc.VectorSubcoreMesh(core_axis_name="c", subcore_axis_name="s"),
    scratch_types=[
        pltpu.VMEM((BLK_PER_W, 128), jnp.int32),
        pltpu.VMEM((128, D_MODEL), jnp.float32),
        pltpu.VMEM((128, D_MODEL), jnp.float32),
        pltpu.VMEM((128, D_MODEL), jnp.float32),
        pltpu.VMEM((128, D_MODEL), jnp.float32),
        pltpu.VMEM((D_MODEL, SKEW), jnp.float32),
        pltpu.VMEM((D_MODEL, SKEW), jnp.float32),
        pltpu.VMEM((D_MODEL, SKEW), jnp.float32),
        pltpu.VMEM((D_MODEL, SKEW), jnp.float32),
        pltpu.SemaphoreType.DMA,
        pltpu.SemaphoreType.DMA,
        pltpu.SemaphoreType.DMA,
        pltpu.SemaphoreType.DMA,
        pltpu.SemaphoreType.DMA,
        pltpu.SemaphoreType.DMA,
        pltpu.SemaphoreType.DMA,
        pltpu.SemaphoreType.DMA,
    ],
    compiler_params=pltpu.CompilerParams(
        use_tc_tiling_on_sc=False, needs_layout_passes=False
    ),
)


@jax.jit
def _run(x, table):
    xph = x.T.reshape(25, 8, NI, 128).transpose(0, 2, 1, 3).reshape(NBLK, 128)
    o = _emb(table, xph)
    o = o.transpose(2, 4, 0, 1, 3)
    return o.reshape(4096, NJ, D_MODEL)


def kernel(x, table):
    return _run(x, table)
